# baseline (speedup 1.0000x reference)
import functools

import jax
import jax.numpy as jnp
from jax import lax
from jax.experimental import pallas as pl
from jax.experimental.pallas import tpu as pltpu
from jax.experimental.pallas import tpu_sc as plsc

V, L, DIM = 4096, 50, 64
LP = L // 2
PAD = 128
B = V * L
NC, NS = 2, 16
NW = NC * NS
ROWS_W = B // NW
CHUNK = 128
HALF = CHUNK // 2
NCHUNK = ROWS_W // CHUNK
NBUF = 5


@functools.lru_cache(maxsize=None)
def _get_sc_gather():
    mesh = plsc.VectorSubcoreMesh(
        core_axis_name="c", subcore_axis_name="s", num_cores=NC, num_subcores=NS
    )

    @functools.partial(
        pl.kernel,
        out_type=jax.ShapeDtypeStruct((B // 2, PAD), jnp.float32),
        mesh=mesh,
        scratch_types=[
            pltpu.VMEM((NCHUNK, CHUNK), jnp.int32),
            [pltpu.VMEM((CHUNK, DIM), jnp.float32) for _ in range(NBUF)],
            [pltpu.SemaphoreType.DMA for _ in range(NBUF)],
        ],
        compiler_params=pltpu.CompilerParams(
            use_tc_tiling_on_sc=False, needs_layout_passes=False
        ),
    )
    def _sc_gather(idx_hbm, table_hbm, out_hbm, idx_v, bufs, sems):
        wid = lax.axis_index("s") * NC + lax.axis_index("c")
        base = wid * ROWS_W
        pltpu.sync_copy(idx_hbm.at[pl.ds(wid * NCHUNK, NCHUNK)], idx_v)

        def start(c, b):
            pltpu.make_async_copy(
                table_hbm.at[idx_v.at[c]], bufs[b], sems[b]
            ).start()

        def wait(c, b):
            pltpu.make_async_copy(
                table_hbm.at[idx_v.at[c]], bufs[b], sems[b]
            ).wait()

        for b in range(NBUF):
            start(b, b)

        @pl.loop(0, NCHUNK, step=NBUF)
        def _(c0):
            for b in range(NBUF):
                c = c0 + b
                wait(c, b)
                orow = pl.ds((base + c * CHUNK) // 2, HALF)
                pltpu.sync_copy(bufs[b].at[pl.ds(0, HALF)],
                                out_hbm.at[orow, pl.ds(0, DIM)])
                pltpu.sync_copy(bufs[b].at[pl.ds(HALF, HALF)],
                                out_hbm.at[orow, pl.ds(DIM, DIM)])

                @pl.when(c + NBUF < NCHUNK)
                def _():
                    start(c + NBUF, b)

    return _sc_gather


VOCAB = 1000000
SPLIT = 512000
WCW = 16000


def _widen_body(a_ref, b_ref, out_ref):
    ii = lax.broadcasted_iota(jnp.int32, (DIM, DIM), 0)
    jj = lax.broadcasted_iota(jnp.int32, (DIM, DIM), 1)
    eye = (ii == jj).astype(jnp.float32)
    tr = lambda x: lax.dot_general(x, eye, (((0,), (0,)), ((), ())))
    out_ref[...] = jnp.concatenate([tr(a_ref[...]), tr(b_ref[...])], axis=1)


_widen = pl.pallas_call(
    _widen_body,
    grid=(SPLIT // WCW,),
    in_specs=[
        pl.BlockSpec((DIM, WCW), lambda i: (0, i)),
        pl.BlockSpec(
            (DIM, WCW),
            lambda i: (0, jnp.minimum(i + SPLIT // WCW, VOCAB // WCW)),
        ),
    ],
    out_specs=pl.BlockSpec((WCW, PAD), lambda i: (i, 0)),
    out_shape=jax.ShapeDtypeStruct((SPLIT, PAD), jnp.float32),
)


def _silu(x):
    half = 0.5 * x
    return half * jnp.tanh(half) + half


BLK = 1024


def _tc_body(fv_ref, x_ref, W1_ref, b1_ref, W2_ref, b2_ref, A1_ref, a1_ref,
             A2_ref, a2_ref, R1_ref, r1_ref, R2_ref, r2_ref, out_ref):
    W1 = W1_ref[...]
    b1 = b1_ref[...][None, :]
    W2 = W2_ref[...]
    b2 = b2_ref[...][None, :]
    A1 = A1_ref[...]
    a1 = a1_ref[...][None, :]
    A2 = A2_ref[...]
    a2 = a2_ref[...]
    R1 = R1_ref[...]
    r1 = r1_ref[...][None, :]
    R2 = R2_ref[...]
    r2 = r2_ref[...][None, :]
    fvb = fv_ref[...]

    x = x_ref[...].reshape(LP * BLK, PAD)
    h = _silu(jnp.dot(x, W1) + b1)
    h = _silu(jnp.dot(h, W2) + b2)
    t = jnp.tanh(jnp.dot(h, A1) + a1)
    logit = jnp.dot(t, A2) + a2
    logit3 = logit.reshape(LP, BLK, 2)

    masked = [
        jnp.where(fvb[:, 2 * l2:2 * l2 + 2] != 0, logit3[l2],
                  jnp.float32(-1e30))
        for l2 in range(LP)
    ]
    m = masked[0]
    for l2 in range(1, LP):
        m = jnp.maximum(m, masked[l2])
    m = jnp.max(m, axis=1, keepdims=True)
    w = jnp.exp(jnp.stack(masked, axis=0) - m[None])
    s = jnp.sum(jnp.sum(w, axis=0), axis=1, keepdims=True)

    li = lax.broadcasted_iota(jnp.int32, (2, PAD), 1) // DIM
    ri = lax.broadcasted_iota(jnp.int32, (2, PAD), 0)
    sel = (li == ri).astype(jnp.float32)
    w128 = jnp.dot(w.reshape(LP * BLK, 2), sel)
    pooled2 = jnp.sum(w128.reshape(LP, BLK, PAD) * h.reshape(LP, BLK, PAD),
                      axis=0)
    pooled = pooled2[:, :DIM] + pooled2[:, DIM:]

    h_pool = pooled / s
    v = _silu(jnp.dot(h_pool, R1) + r1)
    out_ref[...] = jnp.dot(v, R2) + r2


_full = lambda *shape: pl.BlockSpec(shape, lambda i: (0,) * len(shape))

_tc_encode = pl.pallas_call(
    _tc_body,
    grid=(V // BLK,),
    in_specs=[
        pl.BlockSpec((BLK, L), lambda i: (i, 0)),
        pl.BlockSpec((LP, BLK, PAD), lambda i: (0, i, 0)),
        _full(PAD, PAD),
        _full(PAD),
        _full(PAD, PAD),
        _full(PAD),
        _full(PAD, PAD),
        _full(PAD),
        _full(PAD, 2),
        _full(1, 1),
        _full(DIM, DIM),
        _full(DIM),
        _full(DIM, DIM),
        _full(DIM),
    ],
    out_specs=pl.BlockSpec((BLK, DIM), lambda i: (i, 0)),
    out_shape=jax.ShapeDtypeStruct((V, DIM), jnp.float32),
)


def _pair_chunk_indices(flat_visits):
    fv_t = flat_visits.T
    a = fv_t.reshape(LP, 2, V).transpose(0, 2, 1)
    a = a.reshape(B // CHUNK, HALF, 2).transpose(0, 2, 1)
    v = a.reshape(B // CHUNK, CHUNK)
    return jnp.where(v < SPLIT, 2 * v, 2 * (v - SPLIT) + 1)


def kernel(flat_visits, emb, W1, b1, W2, b2, A1, a1, A2, a2, R1, r1, R2, r2):
    embT = emb.T
    table = _widen(embT, embT).reshape(2 * SPLIT, DIM)
    idx = _pair_chunk_indices(flat_visits)
    gx = _get_sc_gather()(idx, table)
    x = gx.reshape(LP, V, PAD)
    eye2 = jnp.eye(2, dtype=jnp.float32)
    W1b = jnp.kron(eye2, W1)
    W2b = jnp.kron(eye2, W2)
    A1b = jnp.kron(eye2, A1)
    A2b = jnp.kron(eye2, A2)
    return _tc_encode(flat_visits, x, W1b, jnp.tile(b1, 2), W2b,
                      jnp.tile(b2, 2), A1b, jnp.tile(a1, 2), A2b,
                      a2.reshape(1, 1), R1, r1, R2, r2)


# speedup vs baseline: 1.0048x; 1.0048x over previous
import functools

import jax
import jax.numpy as jnp
from jax import lax
from jax.experimental import pallas as pl
from jax.experimental.pallas import tpu as pltpu
from jax.experimental.pallas import tpu_sc as plsc

V, L, DIM = 4096, 50, 64
LP = L // 2
PAD = 128
B = V * L
NC, NS = 2, 16
NW = NC * NS
ROWS_W = B // NW
CHUNK = 128
HALF = CHUNK // 2
NCHUNK = ROWS_W // CHUNK
NBUF = 5


@functools.lru_cache(maxsize=None)
def _get_sc_gather():
    mesh = plsc.VectorSubcoreMesh(
        core_axis_name="c", subcore_axis_name="s", num_cores=NC, num_subcores=NS
    )

    @functools.partial(
        pl.kernel,
        out_type=jax.ShapeDtypeStruct((B // 2, PAD), jnp.float32),
        mesh=mesh,
        scratch_types=[
            pltpu.VMEM((NCHUNK, CHUNK), jnp.int32),
            [pltpu.VMEM((CHUNK, DIM), jnp.float32) for _ in range(NBUF)],
            [pltpu.SemaphoreType.DMA for _ in range(NBUF)],
        ],
        compiler_params=pltpu.CompilerParams(
            use_tc_tiling_on_sc=False, needs_layout_passes=False
        ),
    )
    def _sc_gather(idx_hbm, table_hbm, out_hbm, idx_v, bufs, sems):
        wid = lax.axis_index("s") * NC + lax.axis_index("c")
        base = wid * ROWS_W
        pltpu.sync_copy(idx_hbm.at[pl.ds(wid * NCHUNK, NCHUNK)], idx_v)

        def start(c, b):
            pltpu.make_async_copy(
                table_hbm.at[idx_v.at[c]], bufs[b], sems[b]
            ).start()

        def wait(c, b):
            pltpu.make_async_copy(
                table_hbm.at[idx_v.at[c]], bufs[b], sems[b]
            ).wait()

        for b in range(NBUF):
            start(b, b)

        @pl.loop(0, NCHUNK, step=NBUF)
        def _(c0):
            for b in range(NBUF):
                c = c0 + b
                wait(c, b)
                orow = pl.ds((base + c * CHUNK) // 2, HALF)
                pltpu.sync_copy(bufs[b].at[pl.ds(0, HALF)],
                                out_hbm.at[orow, pl.ds(0, DIM)])
                pltpu.sync_copy(bufs[b].at[pl.ds(HALF, HALF)],
                                out_hbm.at[orow, pl.ds(DIM, DIM)])

                @pl.when(c + NBUF < NCHUNK)
                def _():
                    start(c + NBUF, b)

    return _sc_gather


VOCAB = 1000000
SPLIT = 512000
WCW = 16000


def _widen_body(a_ref, b_ref, out_ref):
    ii = lax.broadcasted_iota(jnp.int32, (DIM, DIM), 0)
    jj = lax.broadcasted_iota(jnp.int32, (DIM, DIM), 1)
    eye = (ii == jj).astype(jnp.float32)
    tr = lambda x: lax.dot_general(x, eye, (((0,), (0,)), ((), ())))
    out_ref[...] = jnp.concatenate([tr(a_ref[...]), tr(b_ref[...])], axis=1)


_widen = pl.pallas_call(
    _widen_body,
    grid=(SPLIT // WCW,),
    in_specs=[
        pl.BlockSpec((DIM, WCW), lambda i: (0, i)),
        pl.BlockSpec(
            (DIM, WCW),
            lambda i: (0, jnp.minimum(i + SPLIT // WCW, VOCAB // WCW)),
        ),
    ],
    out_specs=pl.BlockSpec((WCW, PAD), lambda i: (i, 0)),
    out_shape=jax.ShapeDtypeStruct((SPLIT, PAD), jnp.float32),
)


def _silu(x):
    half = 0.5 * x
    return half * jnp.tanh(half) + half


BLK = 512


def _tc_body(fv_ref, x_ref, W1_ref, b1_ref, W2_ref, b2_ref, A1_ref, a1_ref,
             A2_ref, a2_ref, R1_ref, r1_ref, R2_ref, r2_ref, out_ref):
    W1 = W1_ref[...]
    b1 = b1_ref[...][None, :]
    W2 = W2_ref[...]
    b2 = b2_ref[...][None, :]
    A1 = A1_ref[...]
    a1 = a1_ref[...][None, :]
    A2 = A2_ref[...]
    a2 = a2_ref[...]
    R1 = R1_ref[...]
    r1 = r1_ref[...][None, :]
    R2 = R2_ref[...]
    r2 = r2_ref[...][None, :]
    fvb = fv_ref[...]

    x = x_ref[...].reshape(LP * BLK, PAD)
    h = _silu(jnp.dot(x, W1) + b1)
    h = _silu(jnp.dot(h, W2) + b2)
    t = jnp.tanh(jnp.dot(h, A1) + a1)
    logit = jnp.dot(t, A2) + a2
    logit3 = logit.reshape(LP, BLK, 2)

    masked = [
        jnp.where(fvb[:, 2 * l2:2 * l2 + 2] != 0, logit3[l2],
                  jnp.float32(-1e30))
        for l2 in range(LP)
    ]
    m = masked[0]
    for l2 in range(1, LP):
        m = jnp.maximum(m, masked[l2])
    m = jnp.max(m, axis=1, keepdims=True)
    w = jnp.exp(jnp.stack(masked, axis=0) - m[None])
    s = jnp.sum(jnp.sum(w, axis=0), axis=1, keepdims=True)

    li = lax.broadcasted_iota(jnp.int32, (2, PAD), 1) // DIM
    ri = lax.broadcasted_iota(jnp.int32, (2, PAD), 0)
    sel = (li == ri).astype(jnp.float32)
    w128 = jnp.dot(w.reshape(LP * BLK, 2), sel)
    pooled2 = jnp.sum(w128.reshape(LP, BLK, PAD) * h.reshape(LP, BLK, PAD),
                      axis=0)
    pooled = pooled2[:, :DIM] + pooled2[:, DIM:]

    h_pool = pooled / s
    v = _silu(jnp.dot(h_pool, R1) + r1)
    out_ref[...] = jnp.dot(v, R2) + r2


_full = lambda *shape: pl.BlockSpec(shape, lambda i: (0,) * len(shape))

_tc_encode = pl.pallas_call(
    _tc_body,
    grid=(V // BLK,),
    in_specs=[
        pl.BlockSpec((BLK, L), lambda i: (i, 0)),
        pl.BlockSpec((LP, BLK, PAD), lambda i: (0, i, 0)),
        _full(PAD, PAD),
        _full(PAD),
        _full(PAD, PAD),
        _full(PAD),
        _full(PAD, PAD),
        _full(PAD),
        _full(PAD, 2),
        _full(1, 1),
        _full(DIM, DIM),
        _full(DIM),
        _full(DIM, DIM),
        _full(DIM),
    ],
    out_specs=pl.BlockSpec((BLK, DIM), lambda i: (i, 0)),
    out_shape=jax.ShapeDtypeStruct((V, DIM), jnp.float32),
)


def _pair_chunk_indices(flat_visits):
    fv_t = flat_visits.T
    a = fv_t.reshape(LP, 2, V).transpose(0, 2, 1)
    a = a.reshape(B // CHUNK, HALF, 2).transpose(0, 2, 1)
    v = a.reshape(B // CHUNK, CHUNK)
    return jnp.where(v < SPLIT, 2 * v, 2 * (v - SPLIT) + 1)


def kernel(flat_visits, emb, W1, b1, W2, b2, A1, a1, A2, a2, R1, r1, R2, r2):
    embT = emb.T
    table = _widen(embT, embT).reshape(2 * SPLIT, DIM)
    idx = _pair_chunk_indices(flat_visits)
    gx = _get_sc_gather()(idx, table)
    x = gx.reshape(LP, V, PAD)
    eye2 = jnp.eye(2, dtype=jnp.float32)
    W1b = jnp.kron(eye2, W1)
    W2b = jnp.kron(eye2, W2)
    A1b = jnp.kron(eye2, A1)
    A2b = jnp.kron(eye2, A2)
    return _tc_encode(flat_visits, x, W1b, jnp.tile(b1, 2), W2b,
                      jnp.tile(b2, 2), A1b, jnp.tile(a1, 2), A2b,
                      a2.reshape(1, 1), R1, r1, R2, r2)
